# baseline (speedup 1.0000x reference)
import functools
import jax
import jax.numpy as jnp
from jax import lax
from jax.experimental import pallas as pl
from jax.experimental.pallas import tpu as pltpu
from jax.experimental.pallas import tpu_sc as plsc

_B, _T, _C = 32, 196, 384
_F = _T * _C
_H = _F // 2
_NPAIR = _T // 2
_NBT = _B // 8
_NUNIT = _NPAIR * _NBT
_NW = 32
_KFULL = _NUNIT // _NW
_NTAIL = _NUNIT - _KFULL * _NW
_NVROW = _C // 16


def _make_split():
  info = plsc.get_sparse_core_info()
  num_cores = info.num_cores

  mesh = plsc.VectorSubcoreMesh(core_axis_name="c", subcore_axis_name="s")

  @functools.partial(
      pl.kernel,
      mesh=mesh,
      out_type=jax.ShapeDtypeStruct((_B, 2, _H), jnp.float32),
      scratch_types=[
          pltpu.VMEM((2, 8, _C), jnp.float32),
          pltpu.VMEM((2, 8, _C), jnp.float32),
          pltpu.VMEM((2, 8, _C), jnp.float32),
          pltpu.VMEM((8, 2, _C), jnp.float32),
          pltpu.VMEM((8, 2, _C), jnp.float32),
          pltpu.SemaphoreType.DMA,
          pltpu.SemaphoreType.DMA,
          pltpu.SemaphoreType.DMA,
          pltpu.SemaphoreType.DMA,
          pltpu.SemaphoreType.DMA,
      ],
      compiler_params=pltpu.CompilerParams(
          needs_layout_passes=False,
          disable_bounds_checks=True,
          disable_semaphore_checks=True,
      ),
  )
  def split_kernel(
      xt_hbm, out_hbm, bin0, bin1, btl, bst0, bst1,
      sin0, sin1, stl, sout0, sout1,
  ):
    wid = lax.axis_index("s") * num_cores + lax.axis_index("c")
    lane = lax.iota(jnp.int32, 16)
    p_idx = lane & 1
    c0 = lane >> 1
    bins = (bin0, bin1)
    bsts = (bst0, bst1)
    sin = (sin0, sin1)
    sout = (sout0, sout1)
    b_idx = [jnp.full((16,), br, jnp.int32) for br in range(8)]

    def in_copy(u, b):
      j = u >> 2
      bt = u & 3
      return pltpu.make_async_copy(
          xt_hbm.at[pl.ds(2 * j, 2), pl.ds(8 * bt, 8), :], bins[b], sin[b]
      )

    def out_copy(u, b):
      j = u >> 2
      bt = u & 3
      return pltpu.make_async_copy(
          bsts[b],
          out_hbm.at[pl.ds(8 * bt, 8), :, pl.ds(384 * j, 384)],
          sout[b],
      )

    def compute_unit(bin_c, bst_c):
      @plsc.parallel_loop(0, 16, 1, unroll=8)
      def row_body(q):
        dt = q >> 3
        br = q & 7
        bvec = jnp.full((16,), 0, jnp.int32) + br
        cb = 192 * dt
        for m in range(_NVROW):
          v = bin_c[dt, br, pl.ds(16 * m, 16)]
          plsc.store_scatter(bst_c, [bvec, p_idx, c0 + (cb + 8 * m)], v)

    tail_u = _NW * _KFULL + wid

    def tail_in():
      return pltpu.make_async_copy(
          xt_hbm.at[
              pl.ds(2 * (tail_u >> 2), 2), pl.ds(8 * (tail_u & 3), 8), :
          ],
          btl,
          stl,
      )

    def tail_out():
      return pltpu.make_async_copy(
          bst0,
          out_hbm.at[
              pl.ds(8 * (tail_u & 3), 8), :, pl.ds(384 * (tail_u >> 2), 384)
          ],
          sout0,
      )

    in_copy(wid, 0).start()
    in_copy(wid + _NW, 1).start()

    @pl.when(wid < _NTAIL)
    def _():
      tail_in().start()

    def body(k2, carry):
      for b in range(2):
        u = wid + _NW * (2 * k2 + b)
        in_copy(u, b).wait()

        @pl.when(k2 >= 1)
        def _():
          out_copy(u - 2 * _NW, b).wait()

        compute_unit(bins[b], bsts[b])

        @pl.when(k2 < _KFULL // 2 - 1)
        def _():
          in_copy(u + 2 * _NW, b).start()

        out_copy(u, b).start()
      return carry

    lax.fori_loop(0, _KFULL // 2, body, 0)

    for b in range(2):
      out_copy(wid + _NW * (_KFULL - 2 + b), b).wait()

    @pl.when(wid < _NTAIL)
    def _():
      tail_in().wait()
      compute_unit(btl, bst0)
      tail_out().start()
      tail_out().wait()

  return split_kernel


_split = _make_split()


def kernel(input_tensor):
  xt = jnp.transpose(input_tensor, (1, 0, 2))
  return _split(xt)


# speedup vs baseline: 1.0139x; 1.0139x over previous
import functools
import jax
import jax.numpy as jnp
from jax import lax
from jax.experimental import pallas as pl
from jax.experimental.pallas import tpu as pltpu
from jax.experimental.pallas import tpu_sc as plsc

_B, _T, _C = 32, 196, 384
_F = _T * _C
_H = _F // 2
_NPAIR = _T // 2
_NBT = _B // 8
_NUNIT = _NPAIR * _NBT
_NW = 32
_KFULL = _NUNIT // _NW
_NTAIL = _NUNIT - _KFULL * _NW
_NVROW = _C // 16


def _make_split():
  info = plsc.get_sparse_core_info()
  num_cores = info.num_cores

  mesh = plsc.VectorSubcoreMesh(core_axis_name="c", subcore_axis_name="s")

  @functools.partial(
      pl.kernel,
      mesh=mesh,
      out_type=jax.ShapeDtypeStruct((_B, 2, _H), jnp.float32),
      scratch_types=[
          pltpu.VMEM((2, 8, _C), jnp.float32),
          pltpu.VMEM((2, 8, _C), jnp.float32),
          pltpu.VMEM((2, 8, _C), jnp.float32),
          pltpu.VMEM((8, 2, _C), jnp.float32),
          pltpu.VMEM((8, 2, _C), jnp.float32),
          pltpu.SemaphoreType.DMA,
          pltpu.SemaphoreType.DMA,
          pltpu.SemaphoreType.DMA,
          pltpu.SemaphoreType.DMA,
          pltpu.SemaphoreType.DMA,
      ],
      compiler_params=pltpu.CompilerParams(
          needs_layout_passes=False,
          disable_bounds_checks=True,
          disable_semaphore_checks=True,
      ),
  )
  def split_kernel(
      xt_hbm, out_hbm, bin0, bin1, btl, bst0, bst1,
      sin0, sin1, stl, sout0, sout1,
  ):
    wid = lax.axis_index("s") * num_cores + lax.axis_index("c")
    lane = lax.iota(jnp.int32, 16)
    p_idx = lane & 1
    c0 = lane >> 1
    bins = (bin0, bin1)
    bsts = (bst0, bst1)
    sin = (sin0, sin1)
    sout = (sout0, sout1)
    b_idx = [jnp.full((16,), br, jnp.int32) for br in range(8)]

    def in_copy(u, b):
      j = u >> 2
      bt = u & 3
      return pltpu.make_async_copy(
          xt_hbm.at[pl.ds(2 * j, 2), pl.ds(8 * bt, 8), :], bins[b], sin[b]
      )

    def out_copy(u, b):
      j = u >> 2
      bt = u & 3
      return pltpu.make_async_copy(
          bsts[b],
          out_hbm.at[pl.ds(8 * bt, 8), :, pl.ds(384 * j, 384)],
          sout[b],
      )

    def compute_unit(bin_c, bst_c):
      @plsc.parallel_loop(0, 16, 1, unroll=4)
      def row_body(q):
        dt = q >> 3
        br = q & 7
        bvec = jnp.full((16,), 0, jnp.int32) + br
        cb = 192 * dt
        for m in range(_NVROW):
          v = bin_c[dt, br, pl.ds(16 * m, 16)]
          plsc.store_scatter(bst_c, [bvec, p_idx, c0 + (cb + 8 * m)], v)

    tail_u = _NW * _KFULL + wid

    def tail_in():
      return pltpu.make_async_copy(
          xt_hbm.at[
              pl.ds(2 * (tail_u >> 2), 2), pl.ds(8 * (tail_u & 3), 8), :
          ],
          btl,
          stl,
      )

    def tail_out():
      return pltpu.make_async_copy(
          bst0,
          out_hbm.at[
              pl.ds(8 * (tail_u & 3), 8), :, pl.ds(384 * (tail_u >> 2), 384)
          ],
          sout0,
      )

    in_copy(wid, 0).start()
    in_copy(wid + _NW, 1).start()

    @pl.when(wid < _NTAIL)
    def _():
      tail_in().start()

    def body(k2, carry):
      for b in range(2):
        u = wid + _NW * (2 * k2 + b)
        in_copy(u, b).wait()

        @pl.when(k2 >= 1)
        def _():
          out_copy(u - 2 * _NW, b).wait()

        compute_unit(bins[b], bsts[b])

        @pl.when(k2 < _KFULL // 2 - 1)
        def _():
          in_copy(u + 2 * _NW, b).start()

        out_copy(u, b).start()
      return carry

    lax.fori_loop(0, _KFULL // 2, body, 0)

    for b in range(2):
      out_copy(wid + _NW * (_KFULL - 2 + b), b).wait()

    @pl.when(wid < _NTAIL)
    def _():
      tail_in().wait()
      compute_unit(btl, bst0)
      tail_out().start()
      tail_out().wait()

  return split_kernel


_split = _make_split()


def kernel(input_tensor):
  xt = jnp.transpose(input_tensor, (1, 0, 2))
  return _split(xt)
